# baseline (speedup 1.0000x reference)
import functools

import jax
import jax.numpy as jnp
from jax import lax
from jax.experimental import pallas as pl
from jax.experimental.pallas import tpu as pltpu
from jax.experimental.pallas import tpu_sc as plsc

N = 10000
E = 320000
HID = 16
NPAD = 10240
NC = 2
NS = 16
NW = NC * NS
CH = 80
EPW = E // NW
NCH = EPW // CH
SPW = NPAD // NS
NB = 25
NGRP = NCH // NB

_f32 = jnp.float32
_SDS = jax.ShapeDtypeStruct


def _mesh():
    return plsc.VectorSubcoreMesh(core_axis_name="c", subcore_axis_name="s")


_SC_PARAMS = pltpu.CompilerParams(use_tc_tiling_on_sc=False)


def _sc_pass_a(row3d, col3d, edge_attr, zrow, zcol):
    @functools.partial(
        pl.kernel,
        out_type=(_SDS((NC, NPAD), _f32), _SDS((NC, NPAD, 1), _f32)),
        mesh=_mesh(),
        compiler_params=_SC_PARAMS,
        scratch_types=[
            pltpu.VMEM((NCH, CH), jnp.int32),
            pltpu.VMEM((NCH, CH), jnp.int32),
            pltpu.VMEM((EPW, 1), _f32),
            pltpu.VMEM((CH,), _f32),
            pltpu.VMEM_SHARED((NPAD,), _f32),
            pltpu.VMEM_SHARED((NPAD, 1), _f32),
            pltpu.SemaphoreType.DMA,
        ],
    )
    def k(row_hbm, col_hbm, attr_hbm, z_hbm, zc_hbm,
          outdeg_hbm, outxe_hbm,
          rowbuf, colbuf, attrbuf, onesbuf, deg_acc, xe_acc, sem):
        c = lax.axis_index("c")
        s = lax.axis_index("s")
        wid = c * NS + s
        sl = pl.ds(s * SPW, SPW)
        pltpu.sync_copy(z_hbm, deg_acc.at[sl])
        pltpu.sync_copy(zc_hbm, xe_acc.at[sl, :])
        for i in range(CH // 16):
            onesbuf[pl.ds(i * 16, 16)] = jnp.ones((16,), _f32)
        pltpu.sync_copy(row_hbm.at[wid], rowbuf)
        pltpu.sync_copy(col_hbm.at[wid], colbuf)
        pltpu.sync_copy(attr_hbm.at[pl.ds(wid * EPW, EPW), :], attrbuf)
        plsc.subcore_barrier()

        def fire(j, carry):
            pltpu.async_copy(onesbuf, deg_acc.at[colbuf.at[j]], sem,
                             add=True)
            pltpu.async_copy(attrbuf.at[pl.ds(j * CH, CH), :],
                             xe_acc.at[rowbuf.at[j]], sem, add=True)
            return carry

        lax.fori_loop(0, NCH, fire, 0)

        def drain(j, carry):
            pltpu.make_async_copy(onesbuf, deg_acc.at[colbuf.at[j]],
                                  sem).wait()
            pltpu.make_async_copy(attrbuf.at[pl.ds(j * CH, CH), :],
                                  xe_acc.at[rowbuf.at[j]], sem).wait()
            return carry

        lax.fori_loop(0, NCH, drain, 0)
        plsc.subcore_barrier()
        pltpu.sync_copy(deg_acc.at[sl], outdeg_hbm.at[c, sl])
        pltpu.sync_copy(xe_acc.at[sl, :], outxe_hbm.at[c, sl, :])

    return k(row3d, col3d, edge_attr, zrow, zcol)


def _sc_pass_c(row3d, col3d, g1, zrow16):
    @functools.partial(
        pl.kernel,
        out_type=_SDS((NC, NPAD, HID), _f32),
        mesh=_mesh(),
        compiler_params=_SC_PARAMS,
        scratch_types=[
            pltpu.VMEM((NCH, CH), jnp.int32),
            pltpu.VMEM((NCH, CH), jnp.int32),
            pltpu.VMEM((2 * NB, CH, HID), _f32),
            pltpu.VMEM_SHARED((NPAD, HID), _f32),
            pltpu.SemaphoreType.DMA,
            pltpu.SemaphoreType.DMA,
        ],
    )
    def k(row_hbm, col_hbm, g1_hbm, z_hbm, out_hbm,
          rowbuf, colbuf, vals, acc, gsem, ssem):
        c = lax.axis_index("c")
        s = lax.axis_index("s")
        wid = c * NS + s
        sl = pl.ds(s * SPW, SPW)
        pltpu.sync_copy(z_hbm, acc.at[sl, :])
        pltpu.sync_copy(row_hbm.at[wid], rowbuf)
        pltpu.sync_copy(col_hbm.at[wid], colbuf)
        plsc.subcore_barrier()

        def fire_gathers(g, bank):
            def fg(i, carry):
                pltpu.async_copy(g1_hbm.at[rowbuf.at[g * NB + i]],
                                 vals.at[bank * NB + i], gsem)
                return carry
            lax.fori_loop(0, NB, fg, 0)

        def drain_gathers(g, bank):
            def dg(i, carry):
                pltpu.make_async_copy(g1_hbm.at[rowbuf.at[g * NB + i]],
                                      vals.at[bank * NB + i], gsem).wait()
                return carry
            lax.fori_loop(0, NB, dg, 0)

        def fire_scatters(g, bank):
            def fs(i, carry):
                pltpu.async_copy(vals.at[bank * NB + i],
                                 acc.at[colbuf.at[g * NB + i]], ssem,
                                 add=True)
                return carry
            lax.fori_loop(0, NB, fs, 0)

        def drain_scatters(g, bank):
            def ds(i, carry):
                pltpu.make_async_copy(vals.at[bank * NB + i],
                                      acc.at[colbuf.at[g * NB + i]],
                                      ssem).wait()
                return carry
            lax.fori_loop(0, NB, ds, 0)

        fire_gathers(0, 0)

        def grp(g, carry):
            bank = lax.rem(g, 2)
            drain_gathers(g, bank)

            @pl.when(g < NGRP - 1)
            def _():
                fire_gathers(g + 1, 1 - bank)

            @pl.when(g > 0)
            def _():
                drain_scatters(g - 1, 1 - bank)

            fire_scatters(g, bank)
            return carry

        lax.fori_loop(0, NGRP, grp, 0)
        drain_scatters(NGRP - 1, lax.rem(NGRP - 1, 2))
        plsc.subcore_barrier()
        pltpu.sync_copy(acc.at[sl, :], out_hbm.at[c, sl, :])

    return k(row3d, col3d, g1, zrow16)


def _sc_pass_e(row3d, col3d, g2, zrow):
    @functools.partial(
        pl.kernel,
        out_type=_SDS((NC, NPAD), _f32),
        mesh=_mesh(),
        compiler_params=_SC_PARAMS,
        scratch_types=[
            pltpu.VMEM((NCH, CH), jnp.int32),
            pltpu.VMEM((NCH, CH), jnp.int32),
            pltpu.VMEM((2 * NB, CH), _f32),
            pltpu.VMEM_SHARED((N,), _f32),
            pltpu.VMEM_SHARED((NPAD,), _f32),
            pltpu.SemaphoreType.DMA,
            pltpu.SemaphoreType.DMA,
        ],
    )
    def k(row_hbm, col_hbm, g2_hbm, z_hbm, out_hbm,
          rowbuf, colbuf, vals, g2s, acc, gsem, ssem):
        c = lax.axis_index("c")
        s = lax.axis_index("s")
        wid = c * NS + s
        sl = pl.ds(s * SPW, SPW)
        pltpu.sync_copy(z_hbm, acc.at[sl])

        @pl.when(s == 0)
        def _stage():
            pltpu.sync_copy(g2_hbm, g2s)

        pltpu.sync_copy(row_hbm.at[wid], rowbuf)
        pltpu.sync_copy(col_hbm.at[wid], colbuf)
        plsc.subcore_barrier()

        def fire_gathers(g, bank):
            def fg(i, carry):
                pltpu.async_copy(g2s.at[rowbuf.at[g * NB + i]],
                                 vals.at[bank * NB + i], gsem)
                return carry
            lax.fori_loop(0, NB, fg, 0)

        def drain_gathers(g, bank):
            def dg(i, carry):
                pltpu.make_async_copy(g2s.at[rowbuf.at[g * NB + i]],
                                      vals.at[bank * NB + i], gsem).wait()
                return carry
            lax.fori_loop(0, NB, dg, 0)

        def fire_scatters(g, bank):
            def fs(i, carry):
                pltpu.async_copy(vals.at[bank * NB + i],
                                 acc.at[colbuf.at[g * NB + i]], ssem,
                                 add=True)
                return carry
            lax.fori_loop(0, NB, fs, 0)

        def drain_scatters(g, bank):
            def ds(i, carry):
                pltpu.make_async_copy(vals.at[bank * NB + i],
                                      acc.at[colbuf.at[g * NB + i]],
                                      ssem).wait()
                return carry
            lax.fori_loop(0, NB, ds, 0)

        fire_gathers(0, 0)

        def grp(g, carry):
            bank = lax.rem(g, 2)
            drain_gathers(g, bank)

            @pl.when(g < NGRP - 1)
            def _():
                fire_gathers(g + 1, 1 - bank)

            @pl.when(g > 0)
            def _():
                drain_scatters(g - 1, 1 - bank)

            fire_scatters(g, bank)
            return carry

        lax.fori_loop(0, NGRP, grp, 0)
        drain_scatters(NGRP - 1, lax.rem(NGRP - 1, 2))
        plsc.subcore_barrier()
        pltpu.sync_copy(acc.at[sl], out_hbm.at[c, sl])

    return k(row3d, col3d, g2, zrow)


def _tc_pass_z(edge_index):
    def body(ei, row_ref, col_ref):
        row_ref[...] = ei[0, :]
        col_ref[...] = ei[1, :]

    return pl.pallas_call(
        body,
        out_shape=(_SDS((E,), jnp.int32), _SDS((E,), jnp.int32)),
    )(edge_index)


def _tc_pass_b(x, degp, xep, W1):
    def body(x_ref, dp, xp, w1, g1_ref, dinv_ref):
        deg = 1.0 + dp[0, :N] + dp[1, :N]
        dinv = lax.rsqrt(deg)
        xe2 = xp[0, :N, :] + xp[1, :N, :]
        h1 = jnp.dot(x_ref[...], w1[0:128, :], preferred_element_type=_f32)
        h1 = h1 + xe2 * w1[128:129, :]
        g1_ref[...] = h1 * dinv[:, None]
        dinv_ref[...] = dinv

    return pl.pallas_call(
        body,
        out_shape=(_SDS((N, HID), _f32), _SDS((N,), _f32)),
    )(x, degp, xep, W1)


def _tc_pass_d(s1p, g1, dinv, b1, W2):
    def body(sp, g1r, dv, b1r, w2, g2_ref):
        S = sp[0, :N, :] + sp[1, :N, :] + g1r[...]
        out1 = dv[...][:, None] * S + b1r[...][None, :]
        h = jnp.maximum(out1, 0.0)
        h2 = jnp.dot(h, w2[...], preferred_element_type=_f32)
        g2_ref[...] = h2[:, 0] * dv[...]

    return pl.pallas_call(
        body,
        out_shape=_SDS((N,), _f32),
    )(s1p, g1, dinv, b1, W2)


def _tc_pass_f(s2p, g2, dinv, b2):
    def body(sp, g2r, dv, b2r, out_ref):
        v = dv[...] * (sp[0, :N] + sp[1, :N] + g2r[...])
        out_ref[...] = v[:, None] + b2r[...][None, :]

    return pl.pallas_call(
        body,
        out_shape=_SDS((N, 1), _f32),
    )(s2p, g2, dinv, b2)


def kernel(x, edge_attr, edge_index, W1, b1, W2, b2):
    rowf, colf = _tc_pass_z(edge_index)
    row = rowf.reshape(NW, NCH, CH)
    col = colf.reshape(NW, NCH, CH)
    zrow = jnp.zeros((SPW,), _f32)
    zrow16 = jnp.zeros((SPW, HID), _f32)
    zcol = jnp.zeros((SPW, 1), _f32)

    degp, xep = _sc_pass_a(row, col, edge_attr, zrow, zcol)
    g1, dinv = _tc_pass_b(x, degp, xep, W1)
    s1p = _sc_pass_c(row, col, g1, zrow16)
    g2 = _tc_pass_d(s1p, g1, dinv, b1, W2)
    s2p = _sc_pass_e(row, col, g2, zrow)
    return _tc_pass_f(s2p, g2, dinv, b2)


# speedup vs baseline: 2.8153x; 2.8153x over previous
import functools

import jax
import jax.numpy as jnp
from jax import lax
from jax.experimental import pallas as pl
from jax.experimental.pallas import tpu as pltpu
from jax.experimental.pallas import tpu_sc as plsc

N = 10000
E = 320000
HID = 16
NPAD = 10240
NC = 2
NS = 16
NW = NC * NS
CH = 80
EPW = E // NW
NCH = EPW // CH
SPW = NPAD // NS
NB = 25
NGRP = NCH // NB

_f32 = jnp.float32
_SDS = jax.ShapeDtypeStruct


def _mesh():
    return plsc.VectorSubcoreMesh(core_axis_name="c", subcore_axis_name="s")


_SC_PARAMS = pltpu.CompilerParams(use_tc_tiling_on_sc=False)


def _sc_pass_a(row3d, col3d, attr3d, zrow):
    @functools.partial(
        pl.kernel,
        out_type=(_SDS((NC, NPAD), _f32), _SDS((NC, NPAD), _f32)),
        mesh=_mesh(),
        compiler_params=_SC_PARAMS,
        scratch_types=[
            pltpu.VMEM((NCH, CH), jnp.int32),
            pltpu.VMEM((NCH, CH), jnp.int32),
            pltpu.VMEM((NCH, CH), _f32),
            pltpu.VMEM((CH,), _f32),
            pltpu.VMEM_SHARED((NPAD,), _f32),
            pltpu.VMEM_SHARED((NPAD,), _f32),
            pltpu.SemaphoreType.DMA,
        ],
    )
    def k(row_hbm, col_hbm, attr_hbm, z_hbm,
          outdeg_hbm, outxe_hbm,
          rowbuf, colbuf, attrbuf, onesbuf, deg_acc, xe_acc, sem):
        c = lax.axis_index("c")
        s = lax.axis_index("s")
        wid = c * NS + s
        sl = pl.ds(s * SPW, SPW)
        pltpu.sync_copy(z_hbm, deg_acc.at[sl])
        pltpu.sync_copy(z_hbm, xe_acc.at[sl])
        for i in range(CH // 16):
            onesbuf[pl.ds(i * 16, 16)] = jnp.ones((16,), _f32)
        pltpu.sync_copy(row_hbm.at[wid], rowbuf)
        pltpu.sync_copy(col_hbm.at[wid], colbuf)
        pltpu.sync_copy(attr_hbm.at[wid], attrbuf)
        plsc.subcore_barrier()

        def fire(j, carry):
            pltpu.async_copy(onesbuf, deg_acc.at[colbuf.at[j]], sem,
                             add=True)
            pltpu.async_copy(attrbuf.at[j], xe_acc.at[rowbuf.at[j]], sem,
                             add=True)
            return carry

        lax.fori_loop(0, NCH, fire, 0)

        def drain(j, carry):
            pltpu.make_async_copy(onesbuf, deg_acc.at[colbuf.at[j]],
                                  sem).wait()
            pltpu.make_async_copy(attrbuf.at[j], xe_acc.at[rowbuf.at[j]],
                                  sem).wait()
            return carry

        lax.fori_loop(0, NCH, drain, 0)
        plsc.subcore_barrier()
        pltpu.sync_copy(deg_acc.at[sl], outdeg_hbm.at[c, sl])
        pltpu.sync_copy(xe_acc.at[sl], outxe_hbm.at[c, sl])

    return k(row3d, col3d, attr3d, zrow)


def _sc_pass_c(row3d, col3d, g1, zrow16):
    @functools.partial(
        pl.kernel,
        out_type=_SDS((NC, NPAD, HID), _f32),
        mesh=_mesh(),
        compiler_params=_SC_PARAMS,
        scratch_types=[
            pltpu.VMEM((NCH, CH), jnp.int32),
            pltpu.VMEM((NCH, CH), jnp.int32),
            pltpu.VMEM((2 * NB, CH, HID), _f32),
            pltpu.VMEM_SHARED((NPAD, HID), _f32),
            pltpu.SemaphoreType.DMA,
            pltpu.SemaphoreType.DMA,
        ],
    )
    def k(row_hbm, col_hbm, g1_hbm, z_hbm, out_hbm,
          rowbuf, colbuf, vals, acc, gsem, ssem):
        c = lax.axis_index("c")
        s = lax.axis_index("s")
        wid = c * NS + s
        sl = pl.ds(s * SPW, SPW)
        pltpu.sync_copy(z_hbm, acc.at[sl, :])
        pltpu.sync_copy(row_hbm.at[wid], rowbuf)
        pltpu.sync_copy(col_hbm.at[wid], colbuf)
        plsc.subcore_barrier()

        def fire_gathers(g, bank):
            def fg(i, carry):
                pltpu.async_copy(g1_hbm.at[rowbuf.at[g * NB + i]],
                                 vals.at[bank * NB + i], gsem)
                return carry
            lax.fori_loop(0, NB, fg, 0)

        def drain_gathers(g, bank):
            def dg(i, carry):
                pltpu.make_async_copy(g1_hbm.at[rowbuf.at[g * NB + i]],
                                      vals.at[bank * NB + i], gsem).wait()
                return carry
            lax.fori_loop(0, NB, dg, 0)

        def fire_scatters(g, bank):
            def fs(i, carry):
                pltpu.async_copy(vals.at[bank * NB + i],
                                 acc.at[colbuf.at[g * NB + i]], ssem,
                                 add=True)
                return carry
            lax.fori_loop(0, NB, fs, 0)

        def drain_scatters(g, bank):
            def ds(i, carry):
                pltpu.make_async_copy(vals.at[bank * NB + i],
                                      acc.at[colbuf.at[g * NB + i]],
                                      ssem).wait()
                return carry
            lax.fori_loop(0, NB, ds, 0)

        fire_gathers(0, 0)

        def grp(g, carry):
            bank = lax.rem(g, 2)
            drain_gathers(g, bank)

            @pl.when(g < NGRP - 1)
            def _():
                fire_gathers(g + 1, 1 - bank)

            @pl.when(g > 0)
            def _():
                drain_scatters(g - 1, 1 - bank)

            fire_scatters(g, bank)
            return carry

        lax.fori_loop(0, NGRP, grp, 0)
        drain_scatters(NGRP - 1, lax.rem(NGRP - 1, 2))
        plsc.subcore_barrier()
        pltpu.sync_copy(acc.at[sl, :], out_hbm.at[c, sl, :])

    return k(row3d, col3d, g1, zrow16)


def _sc_pass_e(row3d, col3d, g2, zrow):
    @functools.partial(
        pl.kernel,
        out_type=_SDS((NC, NPAD), _f32),
        mesh=_mesh(),
        compiler_params=_SC_PARAMS,
        scratch_types=[
            pltpu.VMEM((NCH, CH), jnp.int32),
            pltpu.VMEM((NCH, CH), jnp.int32),
            pltpu.VMEM((2 * NB, CH), _f32),
            pltpu.VMEM_SHARED((N,), _f32),
            pltpu.VMEM_SHARED((NPAD,), _f32),
            pltpu.SemaphoreType.DMA,
            pltpu.SemaphoreType.DMA,
        ],
    )
    def k(row_hbm, col_hbm, g2_hbm, z_hbm, out_hbm,
          rowbuf, colbuf, vals, g2s, acc, gsem, ssem):
        c = lax.axis_index("c")
        s = lax.axis_index("s")
        wid = c * NS + s
        sl = pl.ds(s * SPW, SPW)
        pltpu.sync_copy(z_hbm, acc.at[sl])

        @pl.when(s == 0)
        def _stage():
            pltpu.sync_copy(g2_hbm, g2s)

        pltpu.sync_copy(row_hbm.at[wid], rowbuf)
        pltpu.sync_copy(col_hbm.at[wid], colbuf)
        plsc.subcore_barrier()

        def fire_gathers(g, bank):
            def fg(i, carry):
                pltpu.async_copy(g2s.at[rowbuf.at[g * NB + i]],
                                 vals.at[bank * NB + i], gsem)
                return carry
            lax.fori_loop(0, NB, fg, 0)

        def drain_gathers(g, bank):
            def dg(i, carry):
                pltpu.make_async_copy(g2s.at[rowbuf.at[g * NB + i]],
                                      vals.at[bank * NB + i], gsem).wait()
                return carry
            lax.fori_loop(0, NB, dg, 0)

        def fire_scatters(g, bank):
            def fs(i, carry):
                pltpu.async_copy(vals.at[bank * NB + i],
                                 acc.at[colbuf.at[g * NB + i]], ssem,
                                 add=True)
                return carry
            lax.fori_loop(0, NB, fs, 0)

        def drain_scatters(g, bank):
            def ds(i, carry):
                pltpu.make_async_copy(vals.at[bank * NB + i],
                                      acc.at[colbuf.at[g * NB + i]],
                                      ssem).wait()
                return carry
            lax.fori_loop(0, NB, ds, 0)

        fire_gathers(0, 0)

        def grp(g, carry):
            bank = lax.rem(g, 2)
            drain_gathers(g, bank)

            @pl.when(g < NGRP - 1)
            def _():
                fire_gathers(g + 1, 1 - bank)

            @pl.when(g > 0)
            def _():
                drain_scatters(g - 1, 1 - bank)

            fire_scatters(g, bank)
            return carry

        lax.fori_loop(0, NGRP, grp, 0)
        drain_scatters(NGRP - 1, lax.rem(NGRP - 1, 2))
        plsc.subcore_barrier()
        pltpu.sync_copy(acc.at[sl], out_hbm.at[c, sl])

    return k(row3d, col3d, g2, zrow)


def _tc_pass_z(edge_index):
    def body(ei, row_ref, col_ref):
        row_ref[...] = ei[0, :]
        col_ref[...] = ei[1, :]

    return pl.pallas_call(
        body,
        out_shape=(_SDS((E,), jnp.int32), _SDS((E,), jnp.int32)),
    )(edge_index)


def _tc_pass_b(x, degp, xep, W1):
    def body(x_ref, dp, xp, w1, g1_ref, dinv_ref):
        deg = 1.0 + dp[0, :N] + dp[1, :N]
        dinv = 1.0 / jnp.sqrt(deg)
        xe = xp[0, :N] + xp[1, :N]
        xin = jnp.concatenate([x_ref[...], xe[:, None]], axis=1)
        h1 = jnp.dot(xin, w1[...], preferred_element_type=_f32)
        g1_ref[...] = h1 * dinv[:, None]
        dinv_ref[...] = dinv

    return pl.pallas_call(
        body,
        out_shape=(_SDS((N, HID), _f32), _SDS((N,), _f32)),
    )(x, degp, xep, W1)


def _tc_pass_d(s1p, g1, dinv, b1, W2):
    def body(sp, g1r, dv, b1r, w2, g2_ref):
        S = sp[0, :N, :] + sp[1, :N, :] + g1r[...]
        out1 = dv[...][:, None] * S + b1r[...][None, :]
        h = jnp.maximum(out1, 0.0)
        h2 = jnp.dot(h, w2[...], preferred_element_type=_f32)
        g2_ref[...] = h2[:, 0] * dv[...]

    return pl.pallas_call(
        body,
        out_shape=_SDS((N,), _f32),
    )(s1p, g1, dinv, b1, W2)


def _tc_pass_f(s2p, g2, dinv, b2):
    def body(sp, g2r, dv, b2r, out_ref):
        v = dv[...] * (sp[0, :N] + sp[1, :N] + g2r[...])
        out_ref[...] = v[:, None] + b2r[...][None, :]

    return pl.pallas_call(
        body,
        out_shape=_SDS((N, 1), _f32),
    )(s2p, g2, dinv, b2)


def kernel(x, edge_attr, edge_index, W1, b1, W2, b2):
    rowf, colf = _tc_pass_z(edge_index)
    row = rowf.reshape(NW, NCH, CH)
    col = colf.reshape(NW, NCH, CH)
    attr = edge_attr.reshape(NW, NCH, CH)
    zrow = jnp.zeros((SPW,), _f32)
    zrow16 = jnp.zeros((SPW, HID), _f32)

    degp, xep = _sc_pass_a(row, col, attr, zrow)
    g1, dinv = _tc_pass_b(x, degp, xep, W1)
    s1p = _sc_pass_c(row, col, g1, zrow16)
    g2 = _tc_pass_d(s1p, g1, dinv, b1, W2)
    s2p = _sc_pass_e(row, col, g2, zrow)
    return _tc_pass_f(s2p, g2, dinv, b2)
